# Initial kernel scaffold; baseline (speedup 1.0000x reference)
import jax
import jax.numpy as jnp
from jax.experimental import pallas as pl


def kernel(row, col, dij, pred_charge, batch):
    raise NotImplementedError("write your pallas kernel here")


# speedup vs baseline: 71.9855x; 71.9855x over previous
import functools
import jax
import jax.numpy as jnp
from jax import lax
from jax.experimental import pallas as pl
from jax.experimental.pallas import tpu as pltpu, tpu_sc as plsc

_ANG = 1e-10
_K_COUL = 8987551792.3
_E_CH = 1.602176634e-19
_EV = 1.602176634e-19
_BETA = 18.7
_R0_ANG = 2.2

_N_NODES = 100000
_N_EDGES = 6400000
_N_GRAPHS = 128

_CE = 0.5 * _K_COUL * _E_CH * _E_CH / (_EV * _ANG)
_DSLOPE = _BETA / _R0_ANG

_LANES = 16
_CH = 1600
_NP = 100096


def _rsqrt16(r2):
    i = lax.bitcast_convert_type(r2, jnp.int32)
    i = jnp.int32(0x5F3759DF) - lax.shift_right_arithmetic(i, 1)
    y = lax.bitcast_convert_type(i, jnp.float32)
    half = r2 * jnp.float32(0.5)
    for _ in range(3):
        y = y * (jnp.float32(1.5) - half * y * y)
    return y


def _make_sc_kernel():
    info = plsc.get_sparse_core_info()
    nc, ns = info.num_cores, info.num_subcores
    nw = nc * ns
    epw = _N_EDGES // nw
    nchunks = epw // _CH
    sl = _NP // ns
    mesh = plsc.VectorSubcoreMesh(core_axis_name="c", subcore_axis_name="s")

    @functools.partial(
        pl.kernel,
        mesh=mesh,
        out_type=jax.ShapeDtypeStruct((nc * 4 * _NP,), jnp.float32),
        scratch_types=[
            pltpu.VMEM((_CH,), jnp.int32),
            pltpu.VMEM((_CH,), jnp.int32),
            pltpu.VMEM((2 * _CH,), jnp.int32),
            pltpu.VMEM((_CH,), jnp.float32),
            pltpu.VMEM((_CH,), jnp.float32),
            pltpu.VMEM((_CH,), jnp.float32),
            pltpu.VMEM((_CH,), jnp.float32),
            pltpu.VMEM((_CH,), jnp.float32),
            pltpu.VMEM((_CH,), jnp.float32),
            pltpu.VMEM((2 * _CH,), jnp.float32),
            pltpu.VMEM((2 * _CH,), jnp.float32),
            pltpu.VMEM((2 * _CH,), jnp.float32),
            pltpu.VMEM((sl,), jnp.float32),
            pltpu.VMEM_SHARED((_NP,), jnp.float32),
            pltpu.VMEM_SHARED((_NP,), jnp.float32),
            pltpu.VMEM_SHARED((_NP,), jnp.float32),
            pltpu.VMEM_SHARED((_NP,), jnp.float32),
            pltpu.SemaphoreType.DMA,
        ],
    )
    def sc_kernel(xs_hbm, ys_hbm, zs_hbm, row_hbm, col_hbm, q_hbm, out_hbm,
                  row_v, col_v, idx2_v, xs_v, ys_v, zs_v, qr_v, qc_v,
                  pe_v, pfx_v, pfy_v, pfz_v, zbuf, acc_e, acc_x, acc_y,
                  acc_z, sem):
        c = lax.axis_index("c")
        s = lax.axis_index("s")
        wid = s * nc + c

        def _zb(i, carry):
            zbuf[pl.ds(i * _LANES, _LANES)] = jnp.zeros((_LANES,), jnp.float32)
            return carry
        lax.fori_loop(0, sl // _LANES, _zb, 0)
        off = s * sl
        pltpu.sync_copy(zbuf, acc_e.at[pl.ds(off, sl)])
        pltpu.sync_copy(zbuf, acc_x.at[pl.ds(off, sl)])
        pltpu.sync_copy(zbuf, acc_y.at[pl.ds(off, sl)])
        pltpu.sync_copy(zbuf, acc_z.at[pl.ds(off, sl)])
        plsc.subcore_barrier()

        def _chunk(ci, carry):
            base = wid * epw + ci * _CH
            pltpu.sync_copy(row_hbm.at[pl.ds(base, _CH)], row_v)
            pltpu.sync_copy(col_hbm.at[pl.ds(base, _CH)], col_v)
            pltpu.sync_copy(row_hbm.at[pl.ds(base, _CH)],
                            idx2_v.at[pl.ds(0, _CH)])
            pltpu.sync_copy(col_hbm.at[pl.ds(base, _CH)],
                            idx2_v.at[pl.ds(_CH, _CH)])
            pltpu.sync_copy(xs_hbm.at[pl.ds(base, _CH)], xs_v)
            pltpu.sync_copy(ys_hbm.at[pl.ds(base, _CH)], ys_v)
            pltpu.sync_copy(zs_hbm.at[pl.ds(base, _CH)], zs_v)
            pltpu.async_copy(q_hbm.at[row_v], qr_v, sem).wait()
            pltpu.async_copy(q_hbm.at[col_v], qc_v, sem).wait()

            def _grp(g, carry2):
                o = g * _LANES
                dx = xs_v[pl.ds(o, _LANES)]
                dy = ys_v[pl.ds(o, _LANES)]
                dz = zs_v[pl.ds(o, _LANES)]
                r2 = dx * dx + dy * dy + dz * dz
                rinv = _rsqrt16(r2)
                r = r2 * rinv
                damp = jnp.exp(jnp.minimum(
                    r * jnp.float32(_DSLOPE) - jnp.float32(_BETA),
                    jnp.float32(0.0)))
                qq = qr_v[pl.ds(o, _LANES)] * qc_v[pl.ds(o, _LANES)]
                e = jnp.float32(_CE) * qq * damp * rinv
                fs = (e + e) * (rinv * rinv)
                fx = dx * fs
                fy = dy * fs
                fz = dz * fs
                pe_v[pl.ds(o, _LANES)] = e
                pfx_v[pl.ds(o, _LANES)] = fx
                pfy_v[pl.ds(o, _LANES)] = fy
                pfz_v[pl.ds(o, _LANES)] = fz
                pfx_v[pl.ds(_CH + o, _LANES)] = -fx
                pfy_v[pl.ds(_CH + o, _LANES)] = -fy
                pfz_v[pl.ds(_CH + o, _LANES)] = -fz
                return carry2
            lax.fori_loop(0, _CH // _LANES, _grp, 0)

            pltpu.sync_copy(pe_v, acc_e.at[row_v], add=True)
            pltpu.sync_copy(pfx_v, acc_x.at[idx2_v], add=True)
            pltpu.sync_copy(pfy_v, acc_y.at[idx2_v], add=True)
            pltpu.sync_copy(pfz_v, acc_z.at[idx2_v], add=True)
            return carry
        lax.fori_loop(0, nchunks, _chunk, 0)

        plsc.subcore_barrier()
        obase = c * (4 * _NP) + off
        for ch, acc in enumerate((acc_e, acc_x, acc_y, acc_z)):
            pltpu.sync_copy(acc.at[pl.ds(off, sl)], zbuf)
            pltpu.sync_copy(zbuf, out_hbm.at[pl.ds(obase + ch * _NP, sl)])

    return sc_kernel, nc


def _tc_finish(partials8, batch3d, nc):
    nblk = _NP // 128

    def body(p_ref, b_ref, e_ref, f_ref):
        i = pl.program_id(0)
        p = p_ref[...]
        f = p[1:4, :]
        e_node = p[0:1, :]
        for cc in range(1, nc):
            f = f + p[4 * cc + 1:4 * cc + 4, :]
            e_node = e_node + p[4 * cc:4 * cc + 1, :]
        f_ref[...] = f
        b = b_ref[0, 0, :].reshape(128, 1)
        onehot = (b == lax.broadcasted_iota(jnp.int32, (128, 128), 1)
                  ).astype(jnp.float32)
        contrib = jnp.dot(e_node, onehot, preferred_element_type=jnp.float32)

        @pl.when(i == 0)
        def _():
            e_ref[...] = jnp.zeros_like(e_ref)
        e_ref[...] += contrib

    return pl.pallas_call(
        body,
        grid=(nblk,),
        in_specs=[
            pl.BlockSpec((4 * nc, 128), lambda i: (0, i)),
            pl.BlockSpec((1, 1, 128), lambda i: (i, 0, 0)),
        ],
        out_specs=[
            pl.BlockSpec((1, 128), lambda i: (0, 0)),
            pl.BlockSpec((3, 128), lambda i: (0, i)),
        ],
        out_shape=[
            jax.ShapeDtypeStruct((1, _N_GRAPHS), jnp.float32),
            jax.ShapeDtypeStruct((3, _NP), jnp.float32),
        ],
    )(partials8, batch3d)


@jax.jit
def kernel(row, col, dij, pred_charge, batch):
    sc_kernel, nc = _make_sc_kernel()
    dij_t = dij.T
    xs, ys, zs = dij_t[0], dij_t[1], dij_t[2]
    partials = sc_kernel(xs, ys, zs, row.astype(jnp.int32),
                         col.astype(jnp.int32), pred_charge)
    p8 = partials.reshape(4 * nc, _NP)
    batch_p = jnp.concatenate(
        [batch.astype(jnp.int32),
         jnp.full((_NP - _N_NODES,), _N_GRAPHS - 1, jnp.int32)]
    ).reshape(_NP // 128, 1, 128)
    energy2d, force_p = _tc_finish(p8, batch_p, nc)
    return energy2d.reshape(_N_GRAPHS), force_p[:, :_N_NODES].T


# speedup vs baseline: 87.4773x; 1.2152x over previous
import functools
import jax
import jax.numpy as jnp
from jax import lax
from jax.experimental import pallas as pl
from jax.experimental.pallas import tpu as pltpu, tpu_sc as plsc

_ANG = 1e-10
_K_COUL = 8987551792.3
_E_CH = 1.602176634e-19
_EV = 1.602176634e-19
_BETA = 18.7
_R0_ANG = 2.2

_N_NODES = 100000
_N_EDGES = 6400000
_N_GRAPHS = 128

_CE = 0.5 * _K_COUL * _E_CH * _E_CH / (_EV * _ANG)
_DSLOPE = _BETA / _R0_ANG

_LANES = 16
_CH = 4000
_NP = 100096


def _rsqrt16(r2):
    i = lax.bitcast_convert_type(r2, jnp.int32)
    i = jnp.int32(0x5F3759DF) - lax.shift_right_arithmetic(i, 1)
    y = lax.bitcast_convert_type(i, jnp.float32)
    half = r2 * jnp.float32(0.5)
    for _ in range(3):
        y = y * (jnp.float32(1.5) - half * y * y)
    return y


def _make_sc_kernel():
    info = plsc.get_sparse_core_info()
    nc, ns = info.num_cores, info.num_subcores
    nw = nc * ns
    epw = _N_EDGES // nw
    nchunks = epw // _CH
    sl = _NP // ns
    mesh = plsc.VectorSubcoreMesh(core_axis_name="c", subcore_axis_name="s")

    @functools.partial(
        pl.kernel,
        mesh=mesh,
        out_type=jax.ShapeDtypeStruct((nc * 4 * _NP,), jnp.float32),
        scratch_types=[
            pltpu.VMEM((_CH,), jnp.int32),
            pltpu.VMEM((_CH,), jnp.int32),
            pltpu.VMEM((2 * _CH,), jnp.int32),
            pltpu.VMEM((_CH,), jnp.float32),
            pltpu.VMEM((_CH,), jnp.float32),
            pltpu.VMEM((_CH,), jnp.float32),
            pltpu.VMEM((_CH,), jnp.float32),
            pltpu.VMEM((_CH,), jnp.float32),
            pltpu.VMEM((_CH,), jnp.float32),
            pltpu.VMEM((2 * _CH,), jnp.float32),
            pltpu.VMEM((2 * _CH,), jnp.float32),
            pltpu.VMEM((2 * _CH,), jnp.float32),
            pltpu.VMEM((sl,), jnp.float32),
            pltpu.VMEM_SHARED((_NP,), jnp.float32),
            pltpu.VMEM_SHARED((_NP,), jnp.float32),
            pltpu.VMEM_SHARED((_NP,), jnp.float32),
            pltpu.VMEM_SHARED((_NP,), jnp.float32),
            pltpu.SemaphoreType.DMA,
        ],
    )
    def sc_kernel(xs_hbm, ys_hbm, zs_hbm, row_hbm, col_hbm, q_hbm, out_hbm,
                  row_v, col_v, idx2_v, xs_v, ys_v, zs_v, qr_v, qc_v,
                  pe_v, pfx_v, pfy_v, pfz_v, zbuf, acc_e, acc_x, acc_y,
                  acc_z, sem):
        c = lax.axis_index("c")
        s = lax.axis_index("s")
        wid = s * nc + c

        def _zb(i, carry):
            zbuf[pl.ds(i * _LANES, _LANES)] = jnp.zeros((_LANES,), jnp.float32)
            return carry
        lax.fori_loop(0, sl // _LANES, _zb, 0)
        off = s * sl
        pltpu.sync_copy(zbuf, acc_e.at[pl.ds(off, sl)])
        pltpu.sync_copy(zbuf, acc_x.at[pl.ds(off, sl)])
        pltpu.sync_copy(zbuf, acc_y.at[pl.ds(off, sl)])
        pltpu.sync_copy(zbuf, acc_z.at[pl.ds(off, sl)])
        plsc.subcore_barrier()

        def _chunk(ci, carry):
            base = wid * epw + ci * _CH
            pltpu.sync_copy(row_hbm.at[pl.ds(base, _CH)], row_v)
            pltpu.sync_copy(col_hbm.at[pl.ds(base, _CH)], col_v)
            pltpu.sync_copy(row_hbm.at[pl.ds(base, _CH)],
                            idx2_v.at[pl.ds(0, _CH)])
            pltpu.sync_copy(col_hbm.at[pl.ds(base, _CH)],
                            idx2_v.at[pl.ds(_CH, _CH)])
            pltpu.sync_copy(xs_hbm.at[pl.ds(base, _CH)], xs_v)
            pltpu.sync_copy(ys_hbm.at[pl.ds(base, _CH)], ys_v)
            pltpu.sync_copy(zs_hbm.at[pl.ds(base, _CH)], zs_v)
            pltpu.async_copy(q_hbm.at[row_v], qr_v, sem).wait()
            pltpu.async_copy(q_hbm.at[col_v], qc_v, sem).wait()

            def _grp(g, carry2):
                o = g * _LANES
                dx = xs_v[pl.ds(o, _LANES)]
                dy = ys_v[pl.ds(o, _LANES)]
                dz = zs_v[pl.ds(o, _LANES)]
                r2 = dx * dx + dy * dy + dz * dz
                rinv = _rsqrt16(r2)
                r = r2 * rinv
                damp = jnp.exp(jnp.minimum(
                    r * jnp.float32(_DSLOPE) - jnp.float32(_BETA),
                    jnp.float32(0.0)))
                qq = qr_v[pl.ds(o, _LANES)] * qc_v[pl.ds(o, _LANES)]
                e = jnp.float32(_CE) * qq * damp * rinv
                fs = (e + e) * (rinv * rinv)
                fx = dx * fs
                fy = dy * fs
                fz = dz * fs
                pe_v[pl.ds(o, _LANES)] = e
                pfx_v[pl.ds(o, _LANES)] = fx
                pfy_v[pl.ds(o, _LANES)] = fy
                pfz_v[pl.ds(o, _LANES)] = fz
                pfx_v[pl.ds(_CH + o, _LANES)] = -fx
                pfy_v[pl.ds(_CH + o, _LANES)] = -fy
                pfz_v[pl.ds(_CH + o, _LANES)] = -fz
                return carry2
            lax.fori_loop(0, _CH // _LANES, _grp, 0)

            pltpu.sync_copy(pe_v, acc_e.at[row_v], add=True)
            pltpu.sync_copy(pfx_v, acc_x.at[idx2_v], add=True)
            pltpu.sync_copy(pfy_v, acc_y.at[idx2_v], add=True)
            pltpu.sync_copy(pfz_v, acc_z.at[idx2_v], add=True)
            return carry
        lax.fori_loop(0, nchunks, _chunk, 0)

        plsc.subcore_barrier()
        obase = c * (4 * _NP) + off
        for ch, acc in enumerate((acc_e, acc_x, acc_y, acc_z)):
            pltpu.sync_copy(acc.at[pl.ds(off, sl)], zbuf)
            pltpu.sync_copy(zbuf, out_hbm.at[pl.ds(obase + ch * _NP, sl)])

    return sc_kernel, nc


def _tc_finish(partials8, batch3d, nc):
    nblk = _NP // 128

    def body(p_ref, b_ref, e_ref, f_ref):
        i = pl.program_id(0)
        p = p_ref[...]
        f = p[1:4, :]
        e_node = p[0:1, :]
        for cc in range(1, nc):
            f = f + p[4 * cc + 1:4 * cc + 4, :]
            e_node = e_node + p[4 * cc:4 * cc + 1, :]
        f_ref[...] = f
        b = b_ref[0, 0, :].reshape(128, 1)
        onehot = (b == lax.broadcasted_iota(jnp.int32, (128, 128), 1)
                  ).astype(jnp.float32)
        contrib = jnp.dot(e_node, onehot, preferred_element_type=jnp.float32)

        @pl.when(i == 0)
        def _():
            e_ref[...] = jnp.zeros_like(e_ref)
        e_ref[...] += contrib

    return pl.pallas_call(
        body,
        grid=(nblk,),
        in_specs=[
            pl.BlockSpec((4 * nc, 128), lambda i: (0, i)),
            pl.BlockSpec((1, 1, 128), lambda i: (i, 0, 0)),
        ],
        out_specs=[
            pl.BlockSpec((1, 128), lambda i: (0, 0)),
            pl.BlockSpec((3, 128), lambda i: (0, i)),
        ],
        out_shape=[
            jax.ShapeDtypeStruct((1, _N_GRAPHS), jnp.float32),
            jax.ShapeDtypeStruct((3, _NP), jnp.float32),
        ],
    )(partials8, batch3d)


@jax.jit
def kernel(row, col, dij, pred_charge, batch):
    sc_kernel, nc = _make_sc_kernel()
    dij_t = dij.T
    xs, ys, zs = dij_t[0], dij_t[1], dij_t[2]
    partials = sc_kernel(xs, ys, zs, row.astype(jnp.int32),
                         col.astype(jnp.int32), pred_charge)
    p8 = partials.reshape(4 * nc, _NP)
    batch_p = jnp.concatenate(
        [batch.astype(jnp.int32),
         jnp.full((_NP - _N_NODES,), _N_GRAPHS - 1, jnp.int32)]
    ).reshape(_NP // 128, 1, 128)
    energy2d, force_p = _tc_finish(p8, batch_p, nc)
    return energy2d.reshape(_N_GRAPHS), force_p[:, :_N_NODES].T


# speedup vs baseline: 98.3331x; 1.1241x over previous
import functools
import jax
import jax.numpy as jnp
from jax import lax
from jax.experimental import pallas as pl
from jax.experimental.pallas import tpu as pltpu, tpu_sc as plsc

_ANG = 1e-10
_K_COUL = 8987551792.3
_E_CH = 1.602176634e-19
_EV = 1.602176634e-19
_BETA = 18.7
_R0_ANG = 2.2

_N_NODES = 100000
_N_EDGES = 6400000
_N_GRAPHS = 128

_CE = 0.5 * _K_COUL * _E_CH * _E_CH / (_EV * _ANG)
_DSLOPE = _BETA / _R0_ANG

_LANES = 16
_CH = 4000
_NP = 100096


def _rsqrt16(r2):
    i = lax.bitcast_convert_type(r2, jnp.int32)
    i = jnp.int32(0x5F3759DF) - lax.shift_right_arithmetic(i, 1)
    y = lax.bitcast_convert_type(i, jnp.float32)
    half = r2 * jnp.float32(0.5)
    for _ in range(3):
        y = y * (jnp.float32(1.5) - half * y * y)
    return y


def _make_sc_kernel():
    info = plsc.get_sparse_core_info()
    nc, ns = info.num_cores, info.num_subcores
    nw = nc * ns
    epw = _N_EDGES // nw
    nchunks = epw // _CH
    sl = _NP // ns
    mesh = plsc.VectorSubcoreMesh(core_axis_name="c", subcore_axis_name="s")

    @functools.partial(
        pl.kernel,
        mesh=mesh,
        out_type=jax.ShapeDtypeStruct((nc * 4 * _NP,), jnp.float32),
        scratch_types=[
            pltpu.VMEM((_CH,), jnp.int32),
            pltpu.VMEM((_CH,), jnp.int32),
            pltpu.VMEM((2 * _CH,), jnp.int32),
            pltpu.VMEM((_CH,), jnp.float32),
            pltpu.VMEM((_CH,), jnp.float32),
            pltpu.VMEM((_CH,), jnp.float32),
            pltpu.VMEM((_CH,), jnp.float32),
            pltpu.VMEM((_CH,), jnp.float32),
            pltpu.VMEM((_CH,), jnp.float32),
            pltpu.VMEM((2 * _CH,), jnp.float32),
            pltpu.VMEM((2 * _CH,), jnp.float32),
            pltpu.VMEM((2 * _CH,), jnp.float32),
            pltpu.VMEM((sl,), jnp.float32),
            pltpu.VMEM_SHARED((_NP,), jnp.float32),
            pltpu.VMEM_SHARED((_NP,), jnp.float32),
            pltpu.VMEM_SHARED((_NP,), jnp.float32),
            pltpu.VMEM_SHARED((_NP,), jnp.float32),
            pltpu.SemaphoreType.DMA,
        ],
    )
    def sc_kernel(xs_hbm, ys_hbm, zs_hbm, row_hbm, col_hbm, q_hbm, out_hbm,
                  row_v, col_v, idx2_v, xs_v, ys_v, zs_v, qr_v, qc_v,
                  pe_v, pfx_v, pfy_v, pfz_v, zbuf, acc_e, acc_x, acc_y,
                  acc_z, sem):
        c = lax.axis_index("c")
        s = lax.axis_index("s")
        wid = s * nc + c

        def _zb(i, carry):
            zbuf[pl.ds(i * _LANES, _LANES)] = jnp.zeros((_LANES,), jnp.float32)
            return carry
        lax.fori_loop(0, sl // _LANES, _zb, 0)
        off = s * sl
        pltpu.sync_copy(zbuf, acc_e.at[pl.ds(off, sl)])
        pltpu.sync_copy(zbuf, acc_x.at[pl.ds(off, sl)])
        pltpu.sync_copy(zbuf, acc_y.at[pl.ds(off, sl)])
        pltpu.sync_copy(zbuf, acc_z.at[pl.ds(off, sl)])
        plsc.subcore_barrier()

        def _chunk(ci, carry):
            base = wid * epw + ci * _CH
            reads = [
                pltpu.async_copy(row_hbm.at[pl.ds(base, _CH)], row_v, sem),
                pltpu.async_copy(col_hbm.at[pl.ds(base, _CH)], col_v, sem),
                pltpu.async_copy(row_hbm.at[pl.ds(base, _CH)],
                                 idx2_v.at[pl.ds(0, _CH)], sem),
                pltpu.async_copy(col_hbm.at[pl.ds(base, _CH)],
                                 idx2_v.at[pl.ds(_CH, _CH)], sem),
                pltpu.async_copy(xs_hbm.at[pl.ds(base, _CH)], xs_v, sem),
                pltpu.async_copy(ys_hbm.at[pl.ds(base, _CH)], ys_v, sem),
                pltpu.async_copy(zs_hbm.at[pl.ds(base, _CH)], zs_v, sem),
            ]
            for h in reads:
                h.wait()
            gathers = [
                pltpu.async_copy(q_hbm.at[row_v], qr_v, sem),
                pltpu.async_copy(q_hbm.at[col_v], qc_v, sem),
            ]
            for h in gathers:
                h.wait()

            def _grp(g, carry2):
                o = g * _LANES
                dx = xs_v[pl.ds(o, _LANES)]
                dy = ys_v[pl.ds(o, _LANES)]
                dz = zs_v[pl.ds(o, _LANES)]
                r2 = dx * dx + dy * dy + dz * dz
                rinv = _rsqrt16(r2)
                r = r2 * rinv
                damp = jnp.exp(jnp.minimum(
                    r * jnp.float32(_DSLOPE) - jnp.float32(_BETA),
                    jnp.float32(0.0)))
                qq = qr_v[pl.ds(o, _LANES)] * qc_v[pl.ds(o, _LANES)]
                e = jnp.float32(_CE) * qq * damp * rinv
                fs = (e + e) * (rinv * rinv)
                fx = dx * fs
                fy = dy * fs
                fz = dz * fs
                pe_v[pl.ds(o, _LANES)] = e
                pfx_v[pl.ds(o, _LANES)] = fx
                pfy_v[pl.ds(o, _LANES)] = fy
                pfz_v[pl.ds(o, _LANES)] = fz
                pfx_v[pl.ds(_CH + o, _LANES)] = -fx
                pfy_v[pl.ds(_CH + o, _LANES)] = -fy
                pfz_v[pl.ds(_CH + o, _LANES)] = -fz
                return carry2
            lax.fori_loop(0, _CH // _LANES, _grp, 0)

            scatters = [
                pltpu.async_copy(pe_v, acc_e.at[row_v], sem, add=True),
                pltpu.async_copy(pfx_v, acc_x.at[idx2_v], sem, add=True),
                pltpu.async_copy(pfy_v, acc_y.at[idx2_v], sem, add=True),
                pltpu.async_copy(pfz_v, acc_z.at[idx2_v], sem, add=True),
            ]
            for h in scatters:
                h.wait()
            return carry
        lax.fori_loop(0, nchunks, _chunk, 0)

        plsc.subcore_barrier()
        obase = c * (4 * _NP) + off
        for ch, acc in enumerate((acc_e, acc_x, acc_y, acc_z)):
            pltpu.sync_copy(acc.at[pl.ds(off, sl)], zbuf)
            pltpu.sync_copy(zbuf, out_hbm.at[pl.ds(obase + ch * _NP, sl)])

    return sc_kernel, nc


def _tc_finish(partials8, batch3d, nc):
    nblk = _NP // 128

    def body(p_ref, b_ref, e_ref, f_ref):
        i = pl.program_id(0)
        p = p_ref[...]
        f = p[1:4, :]
        e_node = p[0:1, :]
        for cc in range(1, nc):
            f = f + p[4 * cc + 1:4 * cc + 4, :]
            e_node = e_node + p[4 * cc:4 * cc + 1, :]
        f_ref[...] = f
        b = b_ref[0, 0, :].reshape(128, 1)
        onehot = (b == lax.broadcasted_iota(jnp.int32, (128, 128), 1)
                  ).astype(jnp.float32)
        contrib = jnp.dot(e_node, onehot, preferred_element_type=jnp.float32)

        @pl.when(i == 0)
        def _():
            e_ref[...] = jnp.zeros_like(e_ref)
        e_ref[...] += contrib

    return pl.pallas_call(
        body,
        grid=(nblk,),
        in_specs=[
            pl.BlockSpec((4 * nc, 128), lambda i: (0, i)),
            pl.BlockSpec((1, 1, 128), lambda i: (i, 0, 0)),
        ],
        out_specs=[
            pl.BlockSpec((1, 128), lambda i: (0, 0)),
            pl.BlockSpec((3, 128), lambda i: (0, i)),
        ],
        out_shape=[
            jax.ShapeDtypeStruct((1, _N_GRAPHS), jnp.float32),
            jax.ShapeDtypeStruct((3, _NP), jnp.float32),
        ],
    )(partials8, batch3d)


@jax.jit
def kernel(row, col, dij, pred_charge, batch):
    sc_kernel, nc = _make_sc_kernel()
    dij_t = dij.T
    partials = sc_kernel(dij_t[0], dij_t[1], dij_t[2], row.astype(jnp.int32),
                         col.astype(jnp.int32), pred_charge)
    p8 = partials.reshape(4 * nc, _NP)
    batch_p = jnp.concatenate(
        [batch.astype(jnp.int32),
         jnp.full((_NP - _N_NODES,), _N_GRAPHS - 1, jnp.int32)]
    ).reshape(_NP // 128, 1, 128)
    energy2d, force_p = _tc_finish(p8, batch_p, nc)
    return energy2d.reshape(_N_GRAPHS), force_p[:, :_N_NODES].T
